# baseline (speedup 1.0000x reference)
import functools

import jax
import jax.numpy as jnp
from jax import lax
from jax.experimental import pallas as pl
from jax.experimental.pallas import tpu as pltpu
from jax.experimental.pallas import tpu_sc as plsc

VOCAB = 30000
EMBED = 128
HIDDEN = 1024
MAX_LEN = 2048
B, L = 4, 2048

_NC, _NS = 2, 16
_NW = _NC * _NS
_LC = L // 2
_N_TOK_C = B * _LC
_TOK_PER_W = _N_TOK_C // _NW


def _sc_gather(table, idx2d):
    mesh = plsc.VectorSubcoreMesh(core_axis_name="c", subcore_axis_name="s")

    @functools.partial(
        pl.kernel,
        mesh=mesh,
        out_type=jax.ShapeDtypeStruct((_N_TOK_C, EMBED), jnp.float32),
        scratch_types=[
            pltpu.VMEM((1, _TOK_PER_W), jnp.int32),
            pltpu.VMEM((_TOK_PER_W, EMBED), jnp.float32),
            pltpu.SemaphoreType.DMA,
        ],
    )
    def gather_k(table_hbm, idx_hbm, out_hbm, idx_v, rows_v, sem):
        wid = lax.axis_index("s") * _NC + lax.axis_index("c")
        pltpu.sync_copy(idx_hbm.at[pl.ds(wid, 1)], idx_v)
        pltpu.async_copy(table_hbm.at[idx_v.at[0]], rows_v, sem).wait()
        pltpu.sync_copy(rows_v, out_hbm.at[pl.ds(wid * _TOK_PER_W, _TOK_PER_W)])

    return gather_k(table, idx2d)


_S = 512
_G = _LC // _S


def _tc_body_half(e_ref, w_ref, b_ref, pos_ref, segf_ref, se_ref, out_ref):
    e2 = e_ref[...].reshape(B * _S, EMBED)
    acc = jnp.dot(e2, w_ref[...], preferred_element_type=jnp.float32)
    acc = acc.reshape(B, _S, HIDDEN)
    se0 = se_ref[0:1, :]
    dse = se_ref[1:2, :] - se0
    base = (b_ref[...] + pos_ref[...] + se0)[None, :, :]
    out_ref[...] = acc + base + segf_ref[...] * dse[None, :, :]


def _tc_body_half_aliased(e_ref, w_ref, b_ref, pos_ref, segf_ref, se_ref,
                          buf_ref, out_ref):
    del buf_ref
    _tc_body_half(e_ref, w_ref, b_ref, pos_ref, segf_ref, se_ref, out_ref)


def _tc_project_half(half, e3, W, b2d, pos_embed, segf3, seg_embed, buf):
    off = half * _G

    in_specs = [
        pl.BlockSpec((B, _S, EMBED), lambda g: (0, g, 0)),
        pl.BlockSpec((EMBED, HIDDEN), lambda g: (0, 0)),
        pl.BlockSpec((1, HIDDEN), lambda g: (0, 0)),
        pl.BlockSpec((_S, HIDDEN), lambda g: (off + g, 0)),
        pl.BlockSpec((B, _S, 1), lambda g: (0, off + g, 0)),
        pl.BlockSpec((2, HIDDEN), lambda g: (0, 0)),
    ]
    args = [e3, W, b2d, pos_embed, segf3, seg_embed]
    body = _tc_body_half
    alias = {}
    if buf is not None:
        in_specs.append(pl.BlockSpec(memory_space=pl.ANY))
        args.append(buf)
        body = _tc_body_half_aliased
        alias = {6: 0}
    return pl.pallas_call(
        body,
        grid=(_G,),
        in_specs=in_specs,
        out_specs=pl.BlockSpec((B, _S, HIDDEN), lambda g: (0, off + g, 0)),
        out_shape=jax.ShapeDtypeStruct((B, L, HIDDEN), jnp.float32),
        input_output_aliases=alias,
    )(*args)


def kernel(x, seg, tok_embed1, W, b, pos_embed, seg_embed):
    x = x.astype(jnp.int32)
    idx_a = x[:, :_LC].reshape(_NW, _TOK_PER_W)
    idx_b = x[:, _LC:].reshape(_NW, _TOK_PER_W)
    e_a = _sc_gather(tok_embed1, idx_a).reshape(B, _LC, EMBED)
    e_b = _sc_gather(tok_embed1, idx_b).reshape(B, _LC, EMBED)
    segf3 = seg.reshape(B, L, 1).astype(jnp.float32)
    b2d = b.reshape(1, HIDDEN)
    buf = _tc_project_half(0, e_a, W, b2d, pos_embed, segf3, seg_embed, None)
    out = _tc_project_half(1, e_b, W, b2d, pos_embed, segf3, seg_embed, buf)
    return out
